# Initial kernel scaffold; baseline (speedup 1.0000x reference)
import jax
import jax.numpy as jnp
from jax.experimental import pallas as pl


def kernel(grasp_type_id, grasp_type_feat_weight):
    raise NotImplementedError("write your pallas kernel here")


# speedup vs baseline: 1.4432x; 1.4432x over previous
import functools

import jax
import jax.numpy as jnp
from jax import lax
from jax.experimental import pallas as pl
from jax.experimental.pallas import tpu as pltpu
from jax.experimental.pallas import tpu_sc as plsc

B = 16384
D = 128
NC = 2
NS = 16
NW = NC * NS
CHUNK = 128
CPW = B // (NW * CHUNK)


def _gather_body(table_hbm, idx_hbm, out_hbm, idx_v, rows_v, sem):
    wid = lax.axis_index("s") * NC + lax.axis_index("c")
    base = wid * CPW

    pltpu.sync_copy(idx_hbm.at[pl.ds(base, CPW)], idx_v)

    copies = [
        pltpu.async_copy(table_hbm.at[idx_v.at[j]], rows_v.at[j], sem)
        for j in range(CPW)
    ]
    for j in range(CPW):
        copies[j].wait()
        pltpu.sync_copy(
            rows_v.at[j], out_hbm.at[pl.ds((base + j) * CHUNK, CHUNK)]
        )


def kernel(grasp_type_id, grasp_type_feat_weight):
    idx = grasp_type_id.astype(jnp.int32).reshape(B // CHUNK, CHUNK)
    mesh = plsc.VectorSubcoreMesh(core_axis_name="c", subcore_axis_name="s")
    run = pl.kernel(
        _gather_body,
        mesh=mesh,
        out_type=jax.ShapeDtypeStruct((B, D), jnp.float32),
        scratch_types=[
            pltpu.VMEM((CPW, CHUNK), jnp.int32),
            pltpu.VMEM((CPW, CHUNK, D), jnp.float32),
            pltpu.SemaphoreType.DMA,
        ],
    )
    return run(grasp_type_feat_weight, idx)


# speedup vs baseline: 1.4502x; 1.0049x over previous
import functools

import jax
import jax.numpy as jnp
from jax import lax
from jax.experimental import pallas as pl
from jax.experimental.pallas import tpu as pltpu
from jax.experimental.pallas import tpu_sc as plsc

B = 16384
D = 128
NC = 2
NS = 16
NW = NC * NS
CHUNK = 128
CPW = B // (NW * CHUNK)


def _gather_body(table_hbm, idx_hbm, out_hbm, idx_v, rows_v, gsem, osem):
    wid = lax.axis_index("s") * NC + lax.axis_index("c")
    base = wid * CPW

    pltpu.sync_copy(idx_hbm.at[pl.ds(base, CPW)], idx_v)

    gathers = [
        pltpu.async_copy(table_hbm.at[idx_v.at[j]], rows_v.at[j], gsem)
        for j in range(CPW)
    ]
    writes = []
    for j in range(CPW):
        gathers[j].wait()
        writes.append(
            pltpu.async_copy(
                rows_v.at[j], out_hbm.at[pl.ds((base + j) * CHUNK, CHUNK)], osem
            )
        )
    for w in writes:
        w.wait()


def kernel(grasp_type_id, grasp_type_feat_weight):
    idx = grasp_type_id.astype(jnp.int32).reshape(B // CHUNK, CHUNK)
    mesh = plsc.VectorSubcoreMesh(core_axis_name="c", subcore_axis_name="s")
    run = pl.kernel(
        _gather_body,
        mesh=mesh,
        out_type=jax.ShapeDtypeStruct((B, D), jnp.float32),
        scratch_types=[
            pltpu.VMEM((CPW, CHUNK), jnp.int32),
            pltpu.VMEM((CPW, CHUNK, D), jnp.float32),
            pltpu.SemaphoreType.DMA,
            pltpu.SemaphoreType.DMA,
        ],
    )
    return run(grasp_type_feat_weight, idx)


# speedup vs baseline: 2.8196x; 1.9443x over previous
import functools

import jax
import jax.numpy as jnp
from jax import lax
from jax.experimental import pallas as pl
from jax.experimental.pallas import tpu as pltpu
from jax.experimental.pallas import tpu_sc as plsc

B = 16384
D = 128
NC = 2
NS = 16
NW = NC * NS
CHUNK = 128
CPW = B // (NW * CHUNK)


def _gather_body(table_hbm, idx_hbm, out_hbm, tbl_s, idx_v, rows_v, gsem, osem):
    sid = lax.axis_index("s")
    wid = sid * NC + lax.axis_index("c")
    base = wid * CPW

    @pl.when(sid == 0)
    def _():
        pltpu.sync_copy(table_hbm, tbl_s)

    pltpu.sync_copy(idx_hbm.at[pl.ds(base, CPW)], idx_v)
    plsc.subcore_barrier()

    gathers = [
        pltpu.async_copy(tbl_s.at[idx_v.at[j]], rows_v.at[j], gsem)
        for j in range(CPW)
    ]
    writes = []
    for j in range(CPW):
        gathers[j].wait()
        writes.append(
            pltpu.async_copy(
                rows_v.at[j], out_hbm.at[pl.ds((base + j) * CHUNK, CHUNK)], osem
            )
        )
    for w in writes:
        w.wait()


def kernel(grasp_type_id, grasp_type_feat_weight):
    idx = grasp_type_id.astype(jnp.int32).reshape(B // CHUNK, CHUNK)
    mesh = plsc.VectorSubcoreMesh(core_axis_name="c", subcore_axis_name="s")
    run = pl.kernel(
        _gather_body,
        mesh=mesh,
        out_type=jax.ShapeDtypeStruct((B, D), jnp.float32),
        scratch_types=[
            pltpu.VMEM_SHARED((40, D), jnp.float32),
            pltpu.VMEM((CPW, CHUNK), jnp.int32),
            pltpu.VMEM((CPW, CHUNK, D), jnp.float32),
            pltpu.SemaphoreType.DMA,
            pltpu.SemaphoreType.DMA,
        ],
    )
    return run(grasp_type_feat_weight, idx)
